# baseline (speedup 1.0000x reference)
import jax
import jax.numpy as jnp
from jax import lax
from jax.experimental import pallas as pl
from jax.experimental.pallas import tpu as pltpu
from jax.experimental.pallas import tpu_sc as plsc

BATCH = 4
TOKENS = 200
DIM = 4096
SC_BATCHES = 2
WPR = 16


def _sc_body(idx_hbm, table_hbm, out_hbm, idx_v, ra, rb, ga, gb, wa, wb):
    wid = lax.axis_index("s") * 2 + lax.axis_index("c")
    b = wid // WPR
    j = wid % WPR
    lo = j < 8
    base = jnp.where(lo, 16 * j, 128 + 8 * (j - 8))
    extra = j == 15

    pltpu.sync_copy(idx_hbm.at[b], idx_v)

    def gather(off, n, rows, sem):
        return pltpu.make_async_copy(
            table_hbm.at[idx_v.at[pl.ds(pl.multiple_of(off, 8), n)]],
            rows, sem)

    def write(off, n, rows, sem):
        return pltpu.make_async_copy(
            rows, out_hbm.at[b, pl.ds(pl.multiple_of(off, 8), n)], sem)

    @pl.when(lo)
    def _():
        gather(base, 16, ra, ga).start()
        gather(base, 16, ra, ga).wait()
        write(base, 16, ra, wa).start()
        write(base, 16, ra, wa).wait()

    @pl.when(jnp.logical_not(lo))
    def _():
        gather(base, 8, rb, gb).start()
        gather(base, 8, rb, gb).wait()
        write(base, 8, rb, wb).start()
        write(base, 8, rb, wb).wait()

    @pl.when(extra)
    def _():
        gather(192, 8, rb, gb).start()
        gather(192, 8, rb, gb).wait()
        write(192, 8, rb, wb).start()
        write(192, 8, rb, wb).wait()


def _tc_body(idx_ref, table_ref, out_ref):
    g = pl.program_id(0)
    idx_all = idx_ref[...]
    sel = lax.broadcasted_iota(jnp.int32, (BATCH, TOKENS), 0) == (g + SC_BATCHES)
    row = jnp.sum(jnp.where(sel, idx_all, 0), axis=0)
    onehot_t = (lax.broadcasted_iota(jnp.int32, (TOKENS, TOKENS), 0)
                == row[None, :]).astype(jnp.float32)
    out_ref[0] = lax.dot_general(
        onehot_t, table_ref[...], (((0,), (0,)), ((), ())),
        preferred_element_type=jnp.float32)


@jax.jit
def kernel(indices, embedding_table):
    idx = indices.astype(jnp.int32)
    mesh = plsc.VectorSubcoreMesh(core_axis_name="c", subcore_axis_name="s")
    sc_out = pl.kernel(
        _sc_body,
        mesh=mesh,
        out_type=jax.ShapeDtypeStruct((SC_BATCHES, TOKENS, DIM), jnp.float32),
        scratch_types=[
            pltpu.VMEM((TOKENS,), jnp.int32),
            pltpu.VMEM((16, DIM), jnp.float32),
            pltpu.VMEM((8, DIM), jnp.float32),
            pltpu.SemaphoreType.DMA,
            pltpu.SemaphoreType.DMA,
            pltpu.SemaphoreType.DMA,
            pltpu.SemaphoreType.DMA,
        ],
    )(idx, embedding_table)

    tc_out = pl.pallas_call(
        _tc_body,
        grid=(BATCH - SC_BATCHES,),
        in_specs=[
            pl.BlockSpec((BATCH, TOKENS), lambda g: (0, 0)),
            pl.BlockSpec((TOKENS, DIM), lambda g: (0, 0)),
        ],
        out_specs=pl.BlockSpec((1, TOKENS, DIM), lambda g: (g, 0, 0)),
        out_shape=jax.ShapeDtypeStruct((BATCH - SC_BATCHES, TOKENS, DIM),
                                       jnp.float32),
    )(idx, embedding_table)

    return jnp.concatenate([sc_out, tc_out], axis=0)


# speedup vs baseline: 1.0107x; 1.0107x over previous
import numpy as np
import jax
import jax.numpy as jnp
from jax import lax
from jax.experimental import pallas as pl
from jax.experimental.pallas import tpu as pltpu
from jax.experimental.pallas import tpu_sc as plsc

DIM = 4096
NW = 32
CHUNK = 8
NCHUNKS = 100
IDX_LOAD = 32

_STARTS = np.array([3 * w + min(w, 4) for w in range(NW)])
_WINDOWS = jnp.asarray(_STARTS[:, None] * CHUNK + np.arange(IDX_LOAD)[None, :])


def _gather_body(idx_hbm, table_hbm, out_hbm, idx_v, rows0, rows1,
                 g0, g1, w0, w1):
    wid = lax.axis_index("s") * 2 + lax.axis_index("c")
    start = 3 * wid + jnp.minimum(wid, 4)
    rows = (rows0, rows1)
    gsem = (g0, g1)
    wsem = (w0, w1)

    pltpu.sync_copy(idx_hbm.at[wid, 0], idx_v)

    def gather(k):
        return pltpu.make_async_copy(
            table_hbm.at[idx_v.at[pl.ds(k * CHUNK, CHUNK)]],
            rows[k % 2], gsem[k % 2])

    def write(k):
        return pltpu.make_async_copy(
            rows[k % 2], out_hbm.at[pl.ds((start + k) * CHUNK, CHUNK)],
            wsem[k % 2])

    gather(0).start()
    gather(1).start()

    gather(0).wait()
    write(0).start()
    write(0).wait()
    gather(2).start()

    gather(1).wait()
    write(1).start()
    write(1).wait()

    @pl.when(wid < 4)
    def _():
        gather(3).start()

    gather(2).wait()
    write(2).start()

    @pl.when(wid < 4)
    def _():
        gather(3).wait()
        write(3).start()

    write(2).wait()

    @pl.when(wid < 4)
    def _():
        write(3).wait()


@jax.jit
def kernel(indices, embedding_table):
    b, t = indices.shape
    n = b * t
    idx_flat = indices.reshape(n).astype(jnp.int32)
    idx_flat = jnp.pad(idx_flat, (0, NW * IDX_LOAD - n))
    idx_rep = jnp.take(idx_flat, _WINDOWS, axis=0).reshape(NW, 1, IDX_LOAD)
    mesh = plsc.VectorSubcoreMesh(core_axis_name="c", subcore_axis_name="s")
    out = pl.kernel(
        _gather_body,
        mesh=mesh,
        out_type=jax.ShapeDtypeStruct((n, DIM), jnp.float32),
        scratch_types=[
            pltpu.VMEM((IDX_LOAD,), jnp.int32),
            pltpu.VMEM((CHUNK, DIM), jnp.float32),
            pltpu.VMEM((CHUNK, DIM), jnp.float32),
            pltpu.SemaphoreType.DMA,
            pltpu.SemaphoreType.DMA,
            pltpu.SemaphoreType.DMA,
            pltpu.SemaphoreType.DMA,
        ],
    )(idx_rep, embedding_table)
    return out.reshape(b, t, DIM)


# speedup vs baseline: 1.2169x; 1.2040x over previous
import jax
import jax.numpy as jnp
from jax import lax
from jax.experimental import pallas as pl
from jax.experimental.pallas import tpu as pltpu
from jax.experimental.pallas import tpu_sc as plsc

DIM = 4096
NW = 32
CHUNK = 8
NCHUNKS = 100
IDX_LOAD = 32


def _gather_body(idx_hbm, table_hbm, out_hbm, idx_v, rows0, rows1,
                 g0, g1, w0, w1):
    wid = lax.axis_index("s") * 2 + lax.axis_index("c")
    start = 3 * wid + jnp.minimum(wid, 4)
    rows = (rows0, rows1)
    gsem = (g0, g1)
    wsem = (w0, w1)

    pltpu.sync_copy(idx_hbm.at[pl.ds(start * CHUNK, IDX_LOAD)], idx_v)

    def gather(k):
        return pltpu.make_async_copy(
            table_hbm.at[idx_v.at[pl.ds(k * CHUNK, CHUNK)]],
            rows[k % 2], gsem[k % 2])

    def write(k):
        return pltpu.make_async_copy(
            rows[k % 2], out_hbm.at[pl.ds((start + k) * CHUNK, CHUNK)],
            wsem[k % 2])

    gather(0).start()
    gather(1).start()

    gather(0).wait()
    write(0).start()
    write(0).wait()
    gather(2).start()

    gather(1).wait()
    write(1).start()
    write(1).wait()

    @pl.when(wid < 4)
    def _():
        gather(3).start()

    gather(2).wait()
    write(2).start()

    @pl.when(wid < 4)
    def _():
        gather(3).wait()
        write(3).start()

    write(2).wait()

    @pl.when(wid < 4)
    def _():
        write(3).wait()


@jax.jit
def kernel(indices, embedding_table):
    b, t = indices.shape
    n = b * t
    idx_flat = indices.reshape(n).astype(jnp.int32)
    idx_flat = jnp.pad(idx_flat, (0, NW * IDX_LOAD - n))
    mesh = plsc.VectorSubcoreMesh(core_axis_name="c", subcore_axis_name="s")
    out = pl.kernel(
        _gather_body,
        mesh=mesh,
        out_type=jax.ShapeDtypeStruct((n, DIM), jnp.float32),
        scratch_types=[
            pltpu.VMEM((IDX_LOAD,), jnp.int32),
            pltpu.VMEM((CHUNK, DIM), jnp.float32),
            pltpu.VMEM((CHUNK, DIM), jnp.float32),
            pltpu.SemaphoreType.DMA,
            pltpu.SemaphoreType.DMA,
            pltpu.SemaphoreType.DMA,
            pltpu.SemaphoreType.DMA,
        ],
    )(idx_flat, embedding_table)
    return out.reshape(b, t, DIM)
